# Initial kernel scaffold; baseline (speedup 1.0000x reference)
import jax
import jax.numpy as jnp
from jax.experimental import pallas as pl


def kernel(atom_x, residue_x, edge_index, W_query, W_key, W_value, W_atom_value, ln_ain_g, ln_ain_b, ln_rin_g, ln_rin_b, ln_aout_g, ln_aout_b, ln_rout_g, ln_rout_b, cm_W1, cm_b1, cm_W2, cm_b2, cm_ln_g, cm_ln_b, rm_W1, rm_b1, rm_W2, rm_b2, rm_ln_g, rm_ln_b):
    raise NotImplementedError("write your pallas kernel here")


# speedup vs baseline: 15.3881x; 15.3881x over previous
import functools

import jax
import jax.numpy as jnp
from jax import lax
from jax.experimental import pallas as pl
from jax.experimental.pallas import tpu as pltpu
from jax.experimental.pallas import tpu_sc as plsc

N_A = 10000
N_R = 10000
E = 160000
C = 256
H = 8
D = 32
T = 0.2

NC = 2
NS = 16
NW = NC * NS

BLK_N = 1000
BLK_E = 2000
KCH = 40


def _ln_rows(x, g, b):
    mu = jnp.mean(x, axis=-1, keepdims=True)
    var = jnp.mean((x - mu) * (x - mu), axis=-1, keepdims=True)
    return (x - mu) * lax.rsqrt(var + 1e-5) * g + b


def _head_expand_mat():
    col = lax.broadcasted_iota(jnp.int32, (H, C), 1) // D
    row = lax.broadcasted_iota(jnp.int32, (H, C), 0)
    return (col == row).astype(jnp.float32)


def _head_sum_mat():
    row = lax.broadcasted_iota(jnp.int32, (C, H), 0) // D
    col = lax.broadcasted_iota(jnp.int32, (C, H), 1)
    return (row == col).astype(jnp.float32)


def _prep_body(ax, rx, wq, wk, wv, wav, ga, ba, gr, br, ta, tr):
    xa = _ln_rows(ax[...], ga[...], ba[...])
    xr = _ln_rows(rx[...], gr[...], br[...])
    q = jnp.dot(xa, wq[...], preferred_element_type=jnp.float32)
    av = jnp.dot(xa, wav[...], preferred_element_type=jnp.float32)
    k = jnp.dot(xr, wk[...], preferred_element_type=jnp.float32)
    v = jnp.dot(xr, wv[...], preferred_element_type=jnp.float32)
    ta[...] = jnp.concatenate([q, av], axis=1)
    tr[...] = jnp.concatenate([k, v], axis=1)


def _score_body(gd, gs, score, m):
    qe = gd[...]
    ke = gs[...]
    s = jnp.dot(qe * ke, _head_sum_mat(),
                preferred_element_type=jnp.float32) * (1.0 / (D ** 0.5))
    score[...] = s
    blk_max = jnp.max(s, axis=0, keepdims=True)

    @pl.when(pl.program_id(0) == 0)
    def _():
        m[...] = blk_max

    @pl.when(pl.program_id(0) != 0)
    def _():
        m[...] = jnp.maximum(m[...], blk_max)


def _weight_body(score, m, vs, avd, wa_ref, wr_ref, exp_ref):
    s = score[...]
    sm = s - m[...]
    ex_a = jnp.exp(sm)
    ex_r = jnp.exp(sm * (1.0 / T))
    exp_ref[...] = jnp.concatenate(
        [ex_a, ex_r, jnp.zeros((s.shape[0], C // 2 - 2 * H), jnp.float32)],
        axis=1)
    bmat = _head_expand_mat()
    wa_ref[...] = vs[...] * jnp.dot(ex_a, bmat,
                                    preferred_element_type=jnp.float32)
    wr_ref[...] = avd[...] * jnp.dot(ex_r, bmat,
                                     preferred_element_type=jnp.float32)


def _final_body(x, o_ref, sseg, g_out, b_out, w1, b1, w2, b2, g_m, b_m,
                out):
    o = o_ref[...]
    s = sseg[...]
    recip = jnp.where(s == 0.0, 0.0, 1.0 / s)
    o = o * jnp.dot(recip, _head_expand_mat(),
                    preferred_element_type=jnp.float32)
    o = _ln_rows(o, g_out[...], b_out[...])
    hcat = jnp.concatenate([x[...], o], axis=1)
    h = jnp.maximum(jnp.dot(hcat, w1[...],
                            preferred_element_type=jnp.float32) + b1[...], 0.0)
    h = jnp.dot(h, w2[...], preferred_element_type=jnp.float32) + b2[...]
    out[...] = _ln_rows(h, g_m[...], b_m[...])


def _sc_gather(ta, tr, dst, src, gd, gs, idx_d, idx_s, rows_d, rows_s,
               sem1, sem2):
    wid = lax.axis_index("s") * NC + lax.axis_index("c")
    epw = E // NW
    base0 = wid * epw

    def body(i, _):
        base = base0 + i * KCH
        pltpu.sync_copy(dst.at[pl.ds(base, KCH)], idx_d)
        pltpu.sync_copy(src.at[pl.ds(base, KCH)], idx_s)
        cp1 = pltpu.async_copy(ta.at[idx_d], rows_d, sem1)
        cp2 = pltpu.async_copy(tr.at[idx_s], rows_s, sem2)
        cp1.wait()
        cp2.wait()
        pltpu.sync_copy(rows_d, gd.at[pl.ds(base, KCH)])
        pltpu.sync_copy(rows_s, gs.at[pl.ds(base, KCH)])
        return 0

    lax.fori_loop(0, epw // KCH, body, 0)


def _sc_scatter(wa, wr, expp, eidx, z128, oa, orr, sex, idx, rows, acc):
    sid = lax.axis_index("s")
    cid = lax.axis_index("c")
    col0 = pl.multiple_of(cid * (C // 2), C // 2)
    ept = E // NS
    npt = 1000
    base0 = sid * ept

    def half_phase(w_hbm, key_row, out_hbm):
        @pl.when(sid == 0)
        def _():
            pltpu.sync_copy(z128, acc)

        plsc.subcore_barrier()

        def body(i, _):
            base = base0 + i * KCH
            pltpu.sync_copy(eidx.at[pl.ds(key_row * E + base, KCH)], idx)
            pltpu.sync_copy(w_hbm.at[pl.ds(base, KCH), pl.ds(col0, C // 2)],
                            rows)
            pltpu.sync_copy(rows, acc.at[idx], add=True)
            return 0

        lax.fori_loop(0, ept // KCH, body, 0)
        plsc.subcore_barrier()

        @pl.when(sid < N_A // npt)
        def _():
            pltpu.sync_copy(acc.at[pl.ds(sid * npt, npt)],
                            out_hbm.at[pl.ds(sid * npt, npt),
                                       pl.ds(col0, C // 2)])

        plsc.subcore_barrier()

    half_phase(wa, 0, oa)
    half_phase(wr, 1, orr)

    @pl.when(sid == 0)
    def _():
        pltpu.sync_copy(z128, acc)

    plsc.subcore_barrier()

    def body(i, _):
        base = base0 + i * KCH
        pltpu.sync_copy(eidx.at[pl.ds(cid * E + base, KCH)], idx)
        pltpu.sync_copy(expp.at[pl.ds(base, KCH)], rows)
        pltpu.sync_copy(rows, acc.at[idx], add=True)
        return 0

    lax.fori_loop(0, ept // KCH, body, 0)
    plsc.subcore_barrier()

    @pl.when(sid < N_A // npt)
    def _():
        pltpu.sync_copy(acc.at[pl.ds(sid * npt, npt)],
                        sex.at[cid, pl.ds(sid * npt, npt)])

    plsc.subcore_barrier()


def kernel(atom_x, residue_x, edge_index, W_query, W_key, W_value,
           W_atom_value, ln_ain_g, ln_ain_b, ln_rin_g, ln_rin_b, ln_aout_g,
           ln_aout_b, ln_rout_g, ln_rout_b, cm_W1, cm_b1, cm_W2, cm_b2,
           cm_ln_g, cm_ln_b, rm_W1, rm_b1, rm_W2, rm_b2, rm_ln_g, rm_ln_b):
    f32 = jnp.float32
    dst = edge_index[0]
    src = edge_index[1]

    def row(v):
        return v.reshape(1, -1)

    n_blocks = N_A // BLK_N
    full = lambda shape: pl.BlockSpec(shape, lambda i: (0, 0))
    ta, tr = pl.pallas_call(
        _prep_body,
        grid=(n_blocks,),
        in_specs=[
            pl.BlockSpec((BLK_N, C), lambda i: (i, 0)),
            pl.BlockSpec((BLK_N, C), lambda i: (i, 0)),
            full((C, C)), full((C, C)), full((C, C)), full((C, C)),
            full((1, C)), full((1, C)), full((1, C)), full((1, C)),
        ],
        out_specs=[
            pl.BlockSpec((BLK_N, 2 * C), lambda i: (i, 0)),
            pl.BlockSpec((BLK_N, 2 * C), lambda i: (i, 0)),
        ],
        out_shape=[
            jax.ShapeDtypeStruct((N_A, 2 * C), f32),
            jax.ShapeDtypeStruct((N_R, 2 * C), f32),
        ],
    )(atom_x, residue_x, W_query, W_key, W_value, W_atom_value,
      row(ln_ain_g), row(ln_ain_b), row(ln_rin_g), row(ln_rin_b))

    mesh = plsc.VectorSubcoreMesh(core_axis_name="c", subcore_axis_name="s")
    gd, gs = pl.kernel(
        _sc_gather,
        out_type=[
            jax.ShapeDtypeStruct((E, 2 * C), f32),
            jax.ShapeDtypeStruct((E, 2 * C), f32),
        ],
        mesh=mesh,
        scratch_types=[
            pltpu.VMEM((KCH,), jnp.int32),
            pltpu.VMEM((KCH,), jnp.int32),
            pltpu.VMEM((KCH, 2 * C), f32),
            pltpu.VMEM((KCH, 2 * C), f32),
            pltpu.SemaphoreType.DMA,
            pltpu.SemaphoreType.DMA,
        ],
    )(ta, tr, dst, src)

    e_blocks = E // BLK_E
    score, m = pl.pallas_call(
        _score_body,
        grid=(e_blocks,),
        in_specs=[
            pl.BlockSpec((BLK_E, C), lambda i: (i, 0)),
            pl.BlockSpec((BLK_E, C), lambda i: (i, 0)),
        ],
        out_specs=[
            pl.BlockSpec((BLK_E, H), lambda i: (i, 0)),
            pl.BlockSpec((1, H), lambda i: (0, 0)),
        ],
        out_shape=[
            jax.ShapeDtypeStruct((E, H), f32),
            jax.ShapeDtypeStruct((1, H), f32),
        ],
    )(gd[:, :C], gs[:, :C])

    wa, wr, expp = pl.pallas_call(
        _weight_body,
        grid=(e_blocks,),
        in_specs=[
            pl.BlockSpec((BLK_E, H), lambda i: (i, 0)),
            pl.BlockSpec((1, H), lambda i: (0, 0)),
            pl.BlockSpec((BLK_E, C), lambda i: (i, 0)),
            pl.BlockSpec((BLK_E, C), lambda i: (i, 0)),
        ],
        out_specs=[
            pl.BlockSpec((BLK_E, C), lambda i: (i, 0)),
            pl.BlockSpec((BLK_E, C), lambda i: (i, 0)),
            pl.BlockSpec((BLK_E, C // 2), lambda i: (i, 0)),
        ],
        out_shape=[
            jax.ShapeDtypeStruct((E, C), f32),
            jax.ShapeDtypeStruct((E, C), f32),
            jax.ShapeDtypeStruct((E, C // 2), f32),
        ],
    )(score, m, gs[:, C:], gd[:, C:])

    z128 = jnp.zeros((N_A, C // 2), f32)
    oa, orr, sex = pl.kernel(
        _sc_scatter,
        out_type=[
            jax.ShapeDtypeStruct((N_A, C), f32),
            jax.ShapeDtypeStruct((N_R, C), f32),
            jax.ShapeDtypeStruct((NC, N_A, C // 2), f32),
        ],
        mesh=mesh,
        scratch_types=[
            pltpu.VMEM((KCH,), jnp.int32),
            pltpu.VMEM((KCH, C // 2), f32),
            pltpu.VMEM_SHARED((N_A, C // 2), f32),
        ],
    )(wa, wr, expp, edge_index.reshape(2 * E), z128)

    return _finish(atom_x, residue_x, oa, orr, sex[0, :, :H],
                   sex[1, :, H:2 * H],
                   ln_aout_g, ln_aout_b, ln_rout_g, ln_rout_b, cm_W1, cm_b1,
                   cm_W2, cm_b2, cm_ln_g, cm_ln_b, rm_W1, rm_b1, rm_W2,
                   rm_b2, rm_ln_g, rm_ln_b)


def _finish(atom_x, residue_x, oa, orr, sab, srb, ln_aout_g,
            ln_aout_b, ln_rout_g, ln_rout_b, cm_W1, cm_b1, cm_W2, cm_b2,
            cm_ln_g, cm_ln_b, rm_W1, rm_b1, rm_W2, rm_b2, rm_ln_g, rm_ln_b):
    f32 = jnp.float32
    n_blocks = N_A // BLK_N
    full = lambda shape: pl.BlockSpec(shape, lambda i: (0, 0))

    def row(v):
        return v.reshape(1, -1)

    def final(x, o, sseg, g_out, b_out, w1, b1, w2, b2, g_m, b_m):
        return pl.pallas_call(
            _final_body,
            grid=(n_blocks,),
            in_specs=[
                pl.BlockSpec((BLK_N, C), lambda i: (i, 0)),
                pl.BlockSpec((BLK_N, C), lambda i: (i, 0)),
                pl.BlockSpec((BLK_N, H), lambda i: (i, 0)),
                full((1, C)), full((1, C)),
                full((2 * C, 2 * C)), full((1, 2 * C)),
                full((2 * C, C)), full((1, C)),
                full((1, C)), full((1, C)),
            ],
            out_specs=pl.BlockSpec((BLK_N, C), lambda i: (i, 0)),
            out_shape=jax.ShapeDtypeStruct((N_A, C), f32),
        )(x, o, sseg, g_out, b_out, w1, b1, w2, b2, g_m, b_m)

    ha = final(atom_x, oa, sab, row(ln_aout_g),
               row(ln_aout_b), cm_W1, row(cm_b1), cm_W2, row(cm_b2),
               row(cm_ln_g), row(cm_ln_b))
    hr = final(residue_x, orr, srb, row(ln_rout_g),
               row(ln_rout_b), rm_W1, row(rm_b1), rm_W2, row(rm_b2),
               row(rm_ln_g), row(rm_ln_b))
    return jnp.concatenate([ha, hr], axis=0)


# speedup vs baseline: 20.6967x; 1.3450x over previous
import functools

import jax
import jax.numpy as jnp
from jax import lax
from jax.experimental import pallas as pl
from jax.experimental.pallas import tpu as pltpu
from jax.experimental.pallas import tpu_sc as plsc

N_A = 10000
N_R = 10000
E = 160000
C = 256
H = 8
D = 32
T = 0.2

NC = 2
NS = 16
NW = NC * NS

BLK_N = 1000
BLK_E = 2000
KG = 40
KS = 80


def _ln_rows(x, g, b):
    mu = jnp.mean(x, axis=-1, keepdims=True)
    var = jnp.mean((x - mu) * (x - mu), axis=-1, keepdims=True)
    return (x - mu) * lax.rsqrt(var + 1e-5) * g + b


def _head_expand_mat():
    col = lax.broadcasted_iota(jnp.int32, (H, C), 1) // D
    row = lax.broadcasted_iota(jnp.int32, (H, C), 0)
    return (col == row).astype(jnp.float32)


def _head_sum_mat():
    row = lax.broadcasted_iota(jnp.int32, (C, H), 0) // D
    col = lax.broadcasted_iota(jnp.int32, (C, H), 1)
    return (row == col).astype(jnp.float32)


def _prep_body(ax, rx, wq, wk, wv, wav, ga, ba, gr, br, ta, tr):
    xa = _ln_rows(ax[...], ga[...], ba[...])
    xr = _ln_rows(rx[...], gr[...], br[...])
    q = jnp.dot(xa, wq[...], preferred_element_type=jnp.float32)
    av = jnp.dot(xa, wav[...], preferred_element_type=jnp.float32)
    k = jnp.dot(xr, wk[...], preferred_element_type=jnp.float32)
    v = jnp.dot(xr, wv[...], preferred_element_type=jnp.float32)
    ta[...] = jnp.concatenate([q, av], axis=1)
    tr[...] = jnp.concatenate([k, v], axis=1)


def _score_body(gd, gs, score, m):
    qe = gd[...]
    ke = gs[...]
    s = jnp.dot(qe * ke, _head_sum_mat(),
                preferred_element_type=jnp.float32) * (1.0 / (D ** 0.5))
    score[...] = s
    blk_max = jnp.max(s, axis=0, keepdims=True)

    @pl.when(pl.program_id(0) == 0)
    def _():
        m[...] = blk_max

    @pl.when(pl.program_id(0) != 0)
    def _():
        m[...] = jnp.maximum(m[...], blk_max)


def _weight_body(score, m, vs, avd, wa_ref, wr_ref, exp_ref):
    s = score[...]
    sm = s - m[...]
    ex_a = jnp.exp(sm)
    ex_r = jnp.exp(sm * (1.0 / T))
    exp_ref[...] = jnp.concatenate(
        [ex_a, ex_r, jnp.zeros((s.shape[0], C // 2 - 2 * H), jnp.float32)],
        axis=1)
    bmat = _head_expand_mat()
    wa_ref[...] = vs[...] * jnp.dot(ex_a, bmat,
                                    preferred_element_type=jnp.float32)
    wr_ref[...] = avd[...] * jnp.dot(ex_r, bmat,
                                     preferred_element_type=jnp.float32)


def _final_body(x, o_ref, sseg, g_out, b_out, w1, b1, w2, b2, g_m, b_m,
                out):
    o = o_ref[...]
    s = sseg[...]
    recip = jnp.where(s == 0.0, 0.0, 1.0 / s)
    o = o * jnp.dot(recip, _head_expand_mat(),
                    preferred_element_type=jnp.float32)
    o = _ln_rows(o, g_out[...], b_out[...])
    hcat = jnp.concatenate([x[...], o], axis=1)
    h = jnp.maximum(jnp.dot(hcat, w1[...],
                            preferred_element_type=jnp.float32) + b1[...], 0.0)
    h = jnp.dot(h, w2[...], preferred_element_type=jnp.float32) + b2[...]
    out[...] = _ln_rows(h, g_m[...], b_m[...])


def _sc_gather(ta, tr, dst, src, gd, gs, idx_v, rows_v, sem1):
    wid = lax.axis_index("s") * NC + lax.axis_index("c")
    epw = E // NW
    base0 = wid * epw

    def make_body(key_hbm, table_hbm, out_hbm):
        def body(i, _):
            base = base0 + i * KG
            pltpu.sync_copy(key_hbm.at[pl.ds(base, KG)], idx_v)
            pltpu.async_copy(table_hbm.at[idx_v], rows_v, sem1).wait()
            pltpu.sync_copy(rows_v, out_hbm.at[pl.ds(base, KG)])
            return 0
        return body

    lax.fori_loop(0, epw // KG, make_body(dst, ta, gd), 0)
    lax.fori_loop(0, epw // KG, make_body(src, tr, gs), 0)


def _sc_scatter(wa, wr, expp, eidx, z128, oa, orr, sex,
                idx2, rows2, acc, sem0, sem1):
    sid = lax.axis_index("s")
    cid = lax.axis_index("c")
    col0 = pl.multiple_of(cid * (C // 2), C // 2)
    ept = E // NS
    npt = 1000
    nb = ept // KS
    base0 = sid * ept
    sems = (sem0, sem1)

    def phase(key_off, load_rows, out_dump):
        @pl.when(sid == 0)
        def _():
            pltpu.sync_copy(z128, acc)

        plsc.subcore_barrier()

        def issue(i, b):
            pltpu.async_copy(eidx.at[pl.ds(key_off(i), KS)],
                             idx2.at[b], sems[b])
            load_rows(i, b)

        def wait(b):
            pltpu.make_async_copy(eidx.at[pl.ds(0, KS)],
                                  idx2.at[b], sems[b]).wait()
            pltpu.make_async_copy(expp.at[pl.ds(0, KS)],
                                  rows2.at[b], sems[b]).wait()

        issue(0, 0)

        def body(j, _):
            for b in range(2):
                i = 2 * j + b

                @pl.when(i + 1 < nb)
                def _():
                    issue(i + 1, 1 - b)

                @pl.when(i < nb)
                def _():
                    wait(b)
                    pltpu.sync_copy(rows2.at[b], acc.at[idx2.at[b]],
                                    add=True)
            return 0

        lax.fori_loop(0, (nb + 1) // 2, body, 0)
        plsc.subcore_barrier()
        out_dump()
        plsc.subcore_barrier()

    def half_rows(w_hbm):
        def load(i, b):
            pltpu.async_copy(
                w_hbm.at[pl.ds(base0 + i * KS, KS), pl.ds(col0, C // 2)],
                rows2.at[b], sems[b])
        return load

    def half_dump(out_hbm):
        def dump():
            @pl.when(sid < N_A // npt)
            def _():
                pltpu.sync_copy(acc.at[pl.ds(sid * npt, npt)],
                                out_hbm.at[pl.ds(sid * npt, npt),
                                           pl.ds(col0, C // 2)])
        return dump

    phase(lambda i: base0 + i * KS, half_rows(wa), half_dump(oa))
    phase(lambda i: E + base0 + i * KS, half_rows(wr), half_dump(orr))

    def ex_rows(i, b):
        pltpu.async_copy(expp.at[pl.ds(base0 + i * KS, KS)],
                         rows2.at[b], sems[b])

    def ex_dump():
        @pl.when(sid < N_A // npt)
        def _():
            pltpu.sync_copy(acc.at[pl.ds(sid * npt, npt)],
                            sex.at[cid, pl.ds(sid * npt, npt)])

    phase(lambda i: cid * E + base0 + i * KS, ex_rows, ex_dump)


def kernel(atom_x, residue_x, edge_index, W_query, W_key, W_value,
           W_atom_value, ln_ain_g, ln_ain_b, ln_rin_g, ln_rin_b, ln_aout_g,
           ln_aout_b, ln_rout_g, ln_rout_b, cm_W1, cm_b1, cm_W2, cm_b2,
           cm_ln_g, cm_ln_b, rm_W1, rm_b1, rm_W2, rm_b2, rm_ln_g, rm_ln_b):
    f32 = jnp.float32
    dst = edge_index[0]
    src = edge_index[1]

    def row(v):
        return v.reshape(1, -1)

    n_blocks = N_A // BLK_N
    full = lambda shape: pl.BlockSpec(shape, lambda i: (0, 0))
    ta, tr = pl.pallas_call(
        _prep_body,
        grid=(n_blocks,),
        in_specs=[
            pl.BlockSpec((BLK_N, C), lambda i: (i, 0)),
            pl.BlockSpec((BLK_N, C), lambda i: (i, 0)),
            full((C, C)), full((C, C)), full((C, C)), full((C, C)),
            full((1, C)), full((1, C)), full((1, C)), full((1, C)),
        ],
        out_specs=[
            pl.BlockSpec((BLK_N, 2 * C), lambda i: (i, 0)),
            pl.BlockSpec((BLK_N, 2 * C), lambda i: (i, 0)),
        ],
        out_shape=[
            jax.ShapeDtypeStruct((N_A, 2 * C), f32),
            jax.ShapeDtypeStruct((N_R, 2 * C), f32),
        ],
    )(atom_x, residue_x, W_query, W_key, W_value, W_atom_value,
      row(ln_ain_g), row(ln_ain_b), row(ln_rin_g), row(ln_rin_b))

    mesh = plsc.VectorSubcoreMesh(core_axis_name="c", subcore_axis_name="s")
    gd, gs = pl.kernel(
        _sc_gather,
        out_type=[
            jax.ShapeDtypeStruct((E, 2 * C), f32),
            jax.ShapeDtypeStruct((E, 2 * C), f32),
        ],
        mesh=mesh,
        scratch_types=[
            pltpu.VMEM((KG,), jnp.int32),
            pltpu.VMEM((KG, 2 * C), f32),
            pltpu.SemaphoreType.DMA,
        ],
    )(ta, tr, dst, src)

    e_blocks = E // BLK_E
    score, m = pl.pallas_call(
        _score_body,
        grid=(e_blocks,),
        in_specs=[
            pl.BlockSpec((BLK_E, C), lambda i: (i, 0)),
            pl.BlockSpec((BLK_E, C), lambda i: (i, 0)),
        ],
        out_specs=[
            pl.BlockSpec((BLK_E, H), lambda i: (i, 0)),
            pl.BlockSpec((1, H), lambda i: (0, 0)),
        ],
        out_shape=[
            jax.ShapeDtypeStruct((E, H), f32),
            jax.ShapeDtypeStruct((1, H), f32),
        ],
    )(gd[:, :C], gs[:, :C])

    wa, wr, expp = pl.pallas_call(
        _weight_body,
        grid=(e_blocks,),
        in_specs=[
            pl.BlockSpec((BLK_E, H), lambda i: (i, 0)),
            pl.BlockSpec((1, H), lambda i: (0, 0)),
            pl.BlockSpec((BLK_E, C), lambda i: (i, 0)),
            pl.BlockSpec((BLK_E, C), lambda i: (i, 0)),
        ],
        out_specs=[
            pl.BlockSpec((BLK_E, C), lambda i: (i, 0)),
            pl.BlockSpec((BLK_E, C), lambda i: (i, 0)),
            pl.BlockSpec((BLK_E, C // 2), lambda i: (i, 0)),
        ],
        out_shape=[
            jax.ShapeDtypeStruct((E, C), f32),
            jax.ShapeDtypeStruct((E, C), f32),
            jax.ShapeDtypeStruct((E, C // 2), f32),
        ],
    )(score, m, gs[:, C:], gd[:, C:])

    z128 = jnp.zeros((N_A, C // 2), f32)
    oa, orr, sex = pl.kernel(
        _sc_scatter,
        out_type=[
            jax.ShapeDtypeStruct((N_A, C), f32),
            jax.ShapeDtypeStruct((N_R, C), f32),
            jax.ShapeDtypeStruct((NC, N_A, C // 2), f32),
        ],
        mesh=mesh,
        scratch_types=[
            pltpu.VMEM((2, KS), jnp.int32),
            pltpu.VMEM((2, KS, C // 2), f32),
            pltpu.VMEM_SHARED((N_A, C // 2), f32),
            pltpu.SemaphoreType.DMA,
            pltpu.SemaphoreType.DMA,
        ],
    )(wa, wr, expp, edge_index.reshape(2 * E), z128)

    return _finish(atom_x, residue_x, oa, orr, sex[0, :, :H],
                   sex[1, :, H:2 * H],
                   ln_aout_g, ln_aout_b, ln_rout_g, ln_rout_b, cm_W1, cm_b1,
                   cm_W2, cm_b2, cm_ln_g, cm_ln_b, rm_W1, rm_b1, rm_W2,
                   rm_b2, rm_ln_g, rm_ln_b)


def _finish(atom_x, residue_x, oa, orr, sab, srb, ln_aout_g,
            ln_aout_b, ln_rout_g, ln_rout_b, cm_W1, cm_b1, cm_W2, cm_b2,
            cm_ln_g, cm_ln_b, rm_W1, rm_b1, rm_W2, rm_b2, rm_ln_g, rm_ln_b):
    f32 = jnp.float32
    n_blocks = N_A // BLK_N
    full = lambda shape: pl.BlockSpec(shape, lambda i: (0, 0))

    def row(v):
        return v.reshape(1, -1)

    def final(x, o, sseg, g_out, b_out, w1, b1, w2, b2, g_m, b_m):
        return pl.pallas_call(
            _final_body,
            grid=(n_blocks,),
            in_specs=[
                pl.BlockSpec((BLK_N, C), lambda i: (i, 0)),
                pl.BlockSpec((BLK_N, C), lambda i: (i, 0)),
                pl.BlockSpec((BLK_N, H), lambda i: (i, 0)),
                full((1, C)), full((1, C)),
                full((2 * C, 2 * C)), full((1, 2 * C)),
                full((2 * C, C)), full((1, C)),
                full((1, C)), full((1, C)),
            ],
            out_specs=pl.BlockSpec((BLK_N, C), lambda i: (i, 0)),
            out_shape=jax.ShapeDtypeStruct((N_A, C), f32),
        )(x, o, sseg, g_out, b_out, w1, b1, w2, b2, g_m, b_m)

    ha = final(atom_x, oa, sab, row(ln_aout_g),
               row(ln_aout_b), cm_W1, row(cm_b1), cm_W2, row(cm_b2),
               row(cm_ln_g), row(cm_ln_b))
    hr = final(residue_x, orr, srb, row(ln_rout_g),
               row(ln_rout_b), rm_W1, row(rm_b1), rm_W2, row(rm_b2),
               row(rm_ln_g), row(rm_ln_b))
    return jnp.concatenate([ha, hr], axis=0)


# speedup vs baseline: 24.0562x; 1.1623x over previous
import functools

import jax
import jax.numpy as jnp
from jax import lax
from jax.experimental import pallas as pl
from jax.experimental.pallas import tpu as pltpu
from jax.experimental.pallas import tpu_sc as plsc

N_A = 10000
N_R = 10000
E = 160000
C = 256
H = 8
D = 32
T = 0.2

NC = 2
NS = 16
NW = NC * NS

BLK_N = 1000
BLK_E = 2000
KG = 40
KS = 80


def _ln_rows(x, g, b):
    mu = jnp.mean(x, axis=-1, keepdims=True)
    var = jnp.mean((x - mu) * (x - mu), axis=-1, keepdims=True)
    return (x - mu) * lax.rsqrt(var + 1e-5) * g + b


def _head_expand_mat():
    col = lax.broadcasted_iota(jnp.int32, (H, C), 1) // D
    row = lax.broadcasted_iota(jnp.int32, (H, C), 0)
    return (col == row).astype(jnp.float32)


def _head_sum_mat():
    row = lax.broadcasted_iota(jnp.int32, (C, H), 0) // D
    col = lax.broadcasted_iota(jnp.int32, (C, H), 1)
    return (row == col).astype(jnp.float32)


def _prep_body(ax, rx, wq, wk, wv, wav, ga, ba, gr, br, ta, tr):
    xa = _ln_rows(ax[...], ga[...], ba[...])
    xr = _ln_rows(rx[...], gr[...], br[...])
    q = jnp.dot(xa, wq[...], preferred_element_type=jnp.float32)
    av = jnp.dot(xa, wav[...], preferred_element_type=jnp.float32)
    k = jnp.dot(xr, wk[...], preferred_element_type=jnp.float32)
    v = jnp.dot(xr, wv[...], preferred_element_type=jnp.float32)
    ta[...] = jnp.concatenate([q, av], axis=1)
    tr[...] = jnp.concatenate([k, v], axis=1)


def _score_body(gd, gs, score, m):
    qe = gd[...]
    ke = gs[...]
    s = jnp.dot(qe * ke, _head_sum_mat(),
                preferred_element_type=jnp.float32) * (1.0 / (D ** 0.5))
    score[...] = s
    blk_max = jnp.max(s, axis=0, keepdims=True)

    @pl.when(pl.program_id(0) == 0)
    def _():
        m[...] = blk_max

    @pl.when(pl.program_id(0) != 0)
    def _():
        m[...] = jnp.maximum(m[...], blk_max)


def _weight_body(score, m, vs, avd, wa_ref, wr_ref, exp_ref):
    s = score[...]
    sm = s - m[...]
    ex_a = jnp.exp(sm)
    ex_r = jnp.exp(sm * (1.0 / T))
    exp_ref[...] = jnp.concatenate(
        [ex_a, ex_r, jnp.zeros((s.shape[0], C // 2 - 2 * H), jnp.float32)],
        axis=1)
    bmat = _head_expand_mat()
    wa_ref[...] = vs[...] * jnp.dot(ex_a, bmat,
                                    preferred_element_type=jnp.float32)
    wr_ref[...] = avd[...] * jnp.dot(ex_r, bmat,
                                     preferred_element_type=jnp.float32)


def _final_body(x, o_ref, sseg, g_out, b_out, w1, b1, w2, b2, g_m, b_m,
                out):
    o = o_ref[...]
    s = sseg[...]
    recip = jnp.where(s == 0.0, 0.0, 1.0 / s)
    o = o * jnp.dot(recip, _head_expand_mat(),
                    preferred_element_type=jnp.float32)
    o = _ln_rows(o, g_out[...], b_out[...])
    hcat = jnp.concatenate([x[...], o], axis=1)
    h = jnp.maximum(jnp.dot(hcat, w1[...],
                            preferred_element_type=jnp.float32) + b1[...], 0.0)
    h = jnp.dot(h, w2[...], preferred_element_type=jnp.float32) + b2[...]
    out[...] = _ln_rows(h, g_m[...], b_m[...])


def _sc_gather(ta, tr, dst, src, gd, gs, idx2, rows2,
               semg0, semg1, semw0, semw1):
    wid = lax.axis_index("s") * NC + lax.axis_index("c")
    epw = E // NW
    nb = epw // KG
    base0 = wid * epw
    semg = (semg0, semg1)
    semw = (semw0, semw1)

    def run(key_hbm, table_hbm, out_hbm):
        def issue(i, b):
            pltpu.sync_copy(key_hbm.at[pl.ds(base0 + i * KG, KG)],
                            idx2.at[b])
            pltpu.async_copy(table_hbm.at[idx2.at[b]], rows2.at[b], semg[b])

        def wait_g(b):
            pltpu.make_async_copy(table_hbm.at[pl.ds(0, KG)],
                                  rows2.at[b], semg[b]).wait()

        def wait_w(b):
            pltpu.make_async_copy(rows2.at[b],
                                  out_hbm.at[pl.ds(0, KG)], semw[b]).wait()

        issue(0, 0)

        def body(j, _):
            for b in range(2):
                i = 2 * j + b

                @pl.when(i + 1 < nb)
                def _():
                    @pl.when(i >= 1)
                    def _():
                        wait_w(1 - b)

                    issue(i + 1, 1 - b)

                @pl.when(i < nb)
                def _():
                    wait_g(b)
                    pltpu.async_copy(
                        rows2.at[b], out_hbm.at[pl.ds(base0 + i * KG, KG)],
                        semw[b])
            return 0

        lax.fori_loop(0, (nb + 1) // 2, body, 0)
        wait_w((nb - 1) % 2)
        wait_w((nb - 2) % 2)

    run(dst, ta, gd)
    run(src, tr, gs)


def _sc_scatter(wa, wr, expp, eidx, z128, oa, orr, sex,
                idx2, rows2, acc, sem0, sem1):
    sid = lax.axis_index("s")
    cid = lax.axis_index("c")
    col0 = pl.multiple_of(cid * (C // 2), C // 2)
    ept = E // NS
    npt = 1000
    nb = ept // KS
    base0 = sid * ept
    sems = (sem0, sem1)

    def phase(key_off, load_rows, out_dump):
        @pl.when(sid == 0)
        def _():
            pltpu.sync_copy(z128, acc)

        plsc.subcore_barrier()

        def issue(i, b):
            pltpu.async_copy(eidx.at[pl.ds(key_off(i), KS)],
                             idx2.at[b], sems[b])
            load_rows(i, b)

        def wait(b):
            pltpu.make_async_copy(eidx.at[pl.ds(0, KS)],
                                  idx2.at[b], sems[b]).wait()
            pltpu.make_async_copy(expp.at[pl.ds(0, KS)],
                                  rows2.at[b], sems[b]).wait()

        issue(0, 0)

        def body(j, _):
            for b in range(2):
                i = 2 * j + b

                @pl.when(i + 1 < nb)
                def _():
                    issue(i + 1, 1 - b)

                @pl.when(i < nb)
                def _():
                    wait(b)
                    pltpu.sync_copy(rows2.at[b], acc.at[idx2.at[b]],
                                    add=True)
            return 0

        lax.fori_loop(0, (nb + 1) // 2, body, 0)
        plsc.subcore_barrier()
        out_dump()
        plsc.subcore_barrier()

    def half_rows(w_hbm):
        def load(i, b):
            pltpu.async_copy(
                w_hbm.at[pl.ds(base0 + i * KS, KS), pl.ds(col0, C // 2)],
                rows2.at[b], sems[b])
        return load

    def half_dump(out_hbm):
        def dump():
            @pl.when(sid < N_A // npt)
            def _():
                pltpu.sync_copy(acc.at[pl.ds(sid * npt, npt)],
                                out_hbm.at[pl.ds(sid * npt, npt),
                                           pl.ds(col0, C // 2)])
        return dump

    phase(lambda i: base0 + i * KS, half_rows(wa), half_dump(oa))
    phase(lambda i: E + base0 + i * KS, half_rows(wr), half_dump(orr))

    def ex_rows(i, b):
        pltpu.async_copy(expp.at[pl.ds(base0 + i * KS, KS)],
                         rows2.at[b], sems[b])

    def ex_dump():
        @pl.when(sid < N_A // npt)
        def _():
            pltpu.sync_copy(acc.at[pl.ds(sid * npt, npt)],
                            sex.at[cid, pl.ds(sid * npt, npt)])

    phase(lambda i: cid * E + base0 + i * KS, ex_rows, ex_dump)


def kernel(atom_x, residue_x, edge_index, W_query, W_key, W_value,
           W_atom_value, ln_ain_g, ln_ain_b, ln_rin_g, ln_rin_b, ln_aout_g,
           ln_aout_b, ln_rout_g, ln_rout_b, cm_W1, cm_b1, cm_W2, cm_b2,
           cm_ln_g, cm_ln_b, rm_W1, rm_b1, rm_W2, rm_b2, rm_ln_g, rm_ln_b):
    f32 = jnp.float32
    dst = edge_index[0]
    src = edge_index[1]

    def row(v):
        return v.reshape(1, -1)

    n_blocks = N_A // BLK_N
    full = lambda shape: pl.BlockSpec(shape, lambda i: (0, 0))
    ta, tr = pl.pallas_call(
        _prep_body,
        grid=(n_blocks,),
        in_specs=[
            pl.BlockSpec((BLK_N, C), lambda i: (i, 0)),
            pl.BlockSpec((BLK_N, C), lambda i: (i, 0)),
            full((C, C)), full((C, C)), full((C, C)), full((C, C)),
            full((1, C)), full((1, C)), full((1, C)), full((1, C)),
        ],
        out_specs=[
            pl.BlockSpec((BLK_N, 2 * C), lambda i: (i, 0)),
            pl.BlockSpec((BLK_N, 2 * C), lambda i: (i, 0)),
        ],
        out_shape=[
            jax.ShapeDtypeStruct((N_A, 2 * C), f32),
            jax.ShapeDtypeStruct((N_R, 2 * C), f32),
        ],
    )(atom_x, residue_x, W_query, W_key, W_value, W_atom_value,
      row(ln_ain_g), row(ln_ain_b), row(ln_rin_g), row(ln_rin_b))

    mesh = plsc.VectorSubcoreMesh(core_axis_name="c", subcore_axis_name="s")
    gd, gs = pl.kernel(
        _sc_gather,
        out_type=[
            jax.ShapeDtypeStruct((E, 2 * C), f32),
            jax.ShapeDtypeStruct((E, 2 * C), f32),
        ],
        mesh=mesh,
        scratch_types=[
            pltpu.VMEM((2, KG), jnp.int32),
            pltpu.VMEM((2, KG, 2 * C), f32),
            pltpu.SemaphoreType.DMA,
            pltpu.SemaphoreType.DMA,
            pltpu.SemaphoreType.DMA,
            pltpu.SemaphoreType.DMA,
        ],
    )(ta, tr, dst, src)

    e_blocks = E // BLK_E
    score, m = pl.pallas_call(
        _score_body,
        grid=(e_blocks,),
        in_specs=[
            pl.BlockSpec((BLK_E, C), lambda i: (i, 0)),
            pl.BlockSpec((BLK_E, C), lambda i: (i, 0)),
        ],
        out_specs=[
            pl.BlockSpec((BLK_E, H), lambda i: (i, 0)),
            pl.BlockSpec((1, H), lambda i: (0, 0)),
        ],
        out_shape=[
            jax.ShapeDtypeStruct((E, H), f32),
            jax.ShapeDtypeStruct((1, H), f32),
        ],
    )(gd[:, :C], gs[:, :C])

    wa, wr, expp = pl.pallas_call(
        _weight_body,
        grid=(e_blocks,),
        in_specs=[
            pl.BlockSpec((BLK_E, H), lambda i: (i, 0)),
            pl.BlockSpec((1, H), lambda i: (0, 0)),
            pl.BlockSpec((BLK_E, C), lambda i: (i, 0)),
            pl.BlockSpec((BLK_E, C), lambda i: (i, 0)),
        ],
        out_specs=[
            pl.BlockSpec((BLK_E, C), lambda i: (i, 0)),
            pl.BlockSpec((BLK_E, C), lambda i: (i, 0)),
            pl.BlockSpec((BLK_E, C // 2), lambda i: (i, 0)),
        ],
        out_shape=[
            jax.ShapeDtypeStruct((E, C), f32),
            jax.ShapeDtypeStruct((E, C), f32),
            jax.ShapeDtypeStruct((E, C // 2), f32),
        ],
    )(score, m, gs[:, C:], gd[:, C:])

    z128 = jnp.zeros((N_A, C // 2), f32)
    oa, orr, sex = pl.kernel(
        _sc_scatter,
        out_type=[
            jax.ShapeDtypeStruct((N_A, C), f32),
            jax.ShapeDtypeStruct((N_R, C), f32),
            jax.ShapeDtypeStruct((NC, N_A, C // 2), f32),
        ],
        mesh=mesh,
        scratch_types=[
            pltpu.VMEM((2, KS), jnp.int32),
            pltpu.VMEM((2, KS, C // 2), f32),
            pltpu.VMEM_SHARED((N_A, C // 2), f32),
            pltpu.SemaphoreType.DMA,
            pltpu.SemaphoreType.DMA,
        ],
    )(wa, wr, expp, edge_index.reshape(2 * E), z128)

    return _finish(atom_x, residue_x, oa, orr, sex[0, :, :H],
                   sex[1, :, H:2 * H],
                   ln_aout_g, ln_aout_b, ln_rout_g, ln_rout_b, cm_W1, cm_b1,
                   cm_W2, cm_b2, cm_ln_g, cm_ln_b, rm_W1, rm_b1, rm_W2,
                   rm_b2, rm_ln_g, rm_ln_b)


def _finish(atom_x, residue_x, oa, orr, sab, srb, ln_aout_g,
            ln_aout_b, ln_rout_g, ln_rout_b, cm_W1, cm_b1, cm_W2, cm_b2,
            cm_ln_g, cm_ln_b, rm_W1, rm_b1, rm_W2, rm_b2, rm_ln_g, rm_ln_b):
    f32 = jnp.float32
    n_blocks = N_A // BLK_N
    full = lambda shape: pl.BlockSpec(shape, lambda i: (0, 0))

    def row(v):
        return v.reshape(1, -1)

    def final(x, o, sseg, g_out, b_out, w1, b1, w2, b2, g_m, b_m):
        return pl.pallas_call(
            _final_body,
            grid=(n_blocks,),
            in_specs=[
                pl.BlockSpec((BLK_N, C), lambda i: (i, 0)),
                pl.BlockSpec((BLK_N, C), lambda i: (i, 0)),
                pl.BlockSpec((BLK_N, H), lambda i: (i, 0)),
                full((1, C)), full((1, C)),
                full((2 * C, 2 * C)), full((1, 2 * C)),
                full((2 * C, C)), full((1, C)),
                full((1, C)), full((1, C)),
            ],
            out_specs=pl.BlockSpec((BLK_N, C), lambda i: (i, 0)),
            out_shape=jax.ShapeDtypeStruct((N_A, C), f32),
        )(x, o, sseg, g_out, b_out, w1, b1, w2, b2, g_m, b_m)

    ha = final(atom_x, oa, sab, row(ln_aout_g),
               row(ln_aout_b), cm_W1, row(cm_b1), cm_W2, row(cm_b2),
               row(cm_ln_g), row(cm_ln_b))
    hr = final(residue_x, orr, srb, row(ln_rout_g),
               row(ln_rout_b), rm_W1, row(rm_b1), rm_W2, row(rm_b2),
               row(rm_ln_g), row(rm_ln_b))
    return jnp.concatenate([ha, hr], axis=0)


# speedup vs baseline: 31.3430x; 1.3029x over previous
import functools

import jax
import jax.numpy as jnp
from jax import lax
from jax.experimental import pallas as pl
from jax.experimental.pallas import tpu as pltpu
from jax.experimental.pallas import tpu_sc as plsc

N_A = 10000
N_R = 10000
E = 160000
C = 256
H = 8
D = 32
T = 0.2

NC = 2
NS = 16
NW = NC * NS

BLK_N = 1000
BLK_E = 2000
KG = 40
KS = 80


def _ln_rows(x, g, b):
    mu = jnp.mean(x, axis=-1, keepdims=True)
    var = jnp.mean((x - mu) * (x - mu), axis=-1, keepdims=True)
    return (x - mu) * lax.rsqrt(var + 1e-5) * g + b


def _head_expand_mat():
    col = lax.broadcasted_iota(jnp.int32, (H, C), 1) // D
    row = lax.broadcasted_iota(jnp.int32, (H, C), 0)
    return (col == row).astype(jnp.float32)


def _head_sum_mat():
    row = lax.broadcasted_iota(jnp.int32, (C, H), 0) // D
    col = lax.broadcasted_iota(jnp.int32, (C, H), 1)
    return (row == col).astype(jnp.float32)


def _prep_body(ax, rx, wq, wk, wv, wav, ga, ba, gr, br, ta, tr):
    xa = _ln_rows(ax[...], ga[...], ba[...])
    xr = _ln_rows(rx[...], gr[...], br[...])
    q = jnp.dot(xa, wq[...], preferred_element_type=jnp.float32)
    av = jnp.dot(xa, wav[...], preferred_element_type=jnp.float32)
    k = jnp.dot(xr, wk[...], preferred_element_type=jnp.float32)
    v = jnp.dot(xr, wv[...], preferred_element_type=jnp.float32)
    ta[...] = jnp.concatenate([q, av], axis=1)
    tr[...] = jnp.concatenate([k, v], axis=1)


def _score_body(gd, gs, score, m):
    qe = gd[...]
    ke = gs[...]
    s = jnp.dot(qe * ke, _head_sum_mat(),
                preferred_element_type=jnp.float32) * (1.0 / (D ** 0.5))
    score[...] = s
    blk_max = jnp.max(s, axis=0, keepdims=True)

    @pl.when(pl.program_id(0) == 0)
    def _():
        m[...] = blk_max

    @pl.when(pl.program_id(0) != 0)
    def _():
        m[...] = jnp.maximum(m[...], blk_max)


def _weight_body(score, m, vs, avd, wa_ref, wr_ref, exp_ref):
    s = score[...]
    sm = s - m[...]
    ex_a = jnp.exp(sm)
    ex_r = jnp.exp(sm * (1.0 / T))
    exp_ref[...] = jnp.concatenate(
        [ex_a, ex_r, jnp.zeros((s.shape[0], C // 2 - 2 * H), jnp.float32)],
        axis=1)
    bmat = _head_expand_mat()
    wa_ref[...] = vs[...] * jnp.dot(ex_a, bmat,
                                    preferred_element_type=jnp.float32)
    wr_ref[...] = avd[...] * jnp.dot(ex_r, bmat,
                                     preferred_element_type=jnp.float32)


def _final_body(x, o_ref, sseg, g_out, b_out, w1, b1, w2, b2, g_m, b_m,
                out):
    o = o_ref[...]
    s = sseg[...]
    recip = jnp.where(s == 0.0, 0.0, 1.0 / s)
    o = o * jnp.dot(recip, _head_expand_mat(),
                    preferred_element_type=jnp.float32)
    o = _ln_rows(o, g_out[...], b_out[...])
    hcat = jnp.concatenate([x[...], o], axis=1)
    h = jnp.maximum(jnp.dot(hcat, w1[...],
                            preferred_element_type=jnp.float32) + b1[...], 0.0)
    h = jnp.dot(h, w2[...], preferred_element_type=jnp.float32) + b2[...]
    out[...] = _ln_rows(h, g_m[...], b_m[...])


def _sc_gather(ta, tr, dst, src, gd, gs, idx2, rows2,
               semg0, semg1, semw0, semw1):
    wid = lax.axis_index("s") * NC + lax.axis_index("c")
    epw = E // NW
    nb = epw // KG
    base0 = wid * epw
    semg = (semg0, semg1)
    semw = (semw0, semw1)

    def run(key_hbm, table_hbm, out_hbm):
        def issue(i, b):
            pltpu.sync_copy(key_hbm.at[pl.ds(base0 + i * KG, KG)],
                            idx2.at[b])
            pltpu.async_copy(table_hbm.at[idx2.at[b]], rows2.at[b], semg[b])

        def wait_g(b):
            pltpu.make_async_copy(table_hbm.at[pl.ds(0, KG)],
                                  rows2.at[b], semg[b]).wait()

        def wait_w(b):
            pltpu.make_async_copy(rows2.at[b],
                                  out_hbm.at[pl.ds(0, KG)], semw[b]).wait()

        issue(0, 0)

        def body(j, _):
            for b in range(2):
                i = 2 * j + b

                @pl.when(i + 1 < nb)
                def _():
                    @pl.when(i >= 1)
                    def _():
                        wait_w(1 - b)

                    issue(i + 1, 1 - b)

                @pl.when(i < nb)
                def _():
                    wait_g(b)
                    pltpu.async_copy(
                        rows2.at[b], out_hbm.at[pl.ds(base0 + i * KG, KG)],
                        semw[b])
            return 0

        lax.fori_loop(0, (nb + 1) // 2, body, 0)
        wait_w((nb - 1) % 2)
        wait_w((nb - 2) % 2)

    run(dst, ta, gd)
    run(src, tr, gs)


def _sc_scatter(wa, wr, expp, eidx, z128, oa, orr, sex,
                idx2, rows2, acc, sem0, sem1):
    sid = lax.axis_index("s")
    cid = lax.axis_index("c")
    col0 = pl.multiple_of(cid * (C // 2), C // 2)
    ept = E // NS
    npt = 1000
    nb = ept // KS
    base0 = sid * ept
    sems = (sem0, sem1)

    def phase(key_off, load_rows, out_dump):
        @pl.when(sid == 0)
        def _():
            pltpu.sync_copy(z128, acc)

        plsc.subcore_barrier()

        def issue(i, b):
            pltpu.async_copy(eidx.at[pl.ds(key_off(i), KS)],
                             idx2.at[b], sems[b])
            load_rows(i, b)

        def wait(b):
            pltpu.make_async_copy(eidx.at[pl.ds(0, KS)],
                                  idx2.at[b], sems[b]).wait()
            pltpu.make_async_copy(expp.at[pl.ds(0, KS)],
                                  rows2.at[b], sems[b]).wait()

        issue(0, 0)

        def body(j, _):
            for b in range(2):
                i = 2 * j + b

                @pl.when(i + 1 < nb)
                def _():
                    issue(i + 1, 1 - b)

                @pl.when(i < nb)
                def _():
                    wait(b)
                    pltpu.sync_copy(rows2.at[b], acc.at[idx2.at[b]],
                                    add=True)
            return 0

        lax.fori_loop(0, (nb + 1) // 2, body, 0)
        plsc.subcore_barrier()
        out_dump()
        plsc.subcore_barrier()

    def half_rows(w_hbm):
        def load(i, b):
            pltpu.async_copy(
                w_hbm.at[pl.ds(base0 + i * KS, KS), pl.ds(col0, C // 2)],
                rows2.at[b], sems[b])
        return load

    def half_dump(out_hbm):
        def dump():
            @pl.when(sid < N_A // npt)
            def _():
                pltpu.sync_copy(acc.at[pl.ds(sid * npt, npt)],
                                out_hbm.at[pl.ds(sid * npt, npt),
                                           pl.ds(col0, C // 2)])
        return dump

    phase(lambda i: base0 + i * KS, half_rows(wa), half_dump(oa))
    phase(lambda i: E + base0 + i * KS, half_rows(wr), half_dump(orr))

    def ex_rows(i, b):
        pltpu.async_copy(expp.at[pl.ds(base0 + i * KS, KS)],
                         rows2.at[b], sems[b])

    def ex_dump():
        @pl.when(sid < N_A // npt)
        def _():
            pltpu.sync_copy(acc.at[pl.ds(sid * npt, npt)],
                            sex.at[cid, pl.ds(sid * npt, npt)])

    phase(lambda i: cid * E + base0 + i * KS, ex_rows, ex_dump)


def kernel(atom_x, residue_x, edge_index, W_query, W_key, W_value,
           W_atom_value, ln_ain_g, ln_ain_b, ln_rin_g, ln_rin_b, ln_aout_g,
           ln_aout_b, ln_rout_g, ln_rout_b, cm_W1, cm_b1, cm_W2, cm_b2,
           cm_ln_g, cm_ln_b, rm_W1, rm_b1, rm_W2, rm_b2, rm_ln_g, rm_ln_b):
    f32 = jnp.float32
    dst = edge_index[0]
    src = edge_index[1]

    def row(v):
        return v.reshape(1, -1)

    n_blocks = N_A // BLK_N
    full = lambda shape: pl.BlockSpec(shape, lambda i: (0, 0))
    ta, tr = pl.pallas_call(
        _prep_body,
        grid=(n_blocks,),
        in_specs=[
            pl.BlockSpec((BLK_N, C), lambda i: (i, 0)),
            pl.BlockSpec((BLK_N, C), lambda i: (i, 0)),
            full((C, C)), full((C, C)), full((C, C)), full((C, C)),
            full((1, C)), full((1, C)), full((1, C)), full((1, C)),
        ],
        out_specs=[
            pl.BlockSpec((BLK_N, 2 * C), lambda i: (i, 0)),
            pl.BlockSpec((BLK_N, 2 * C), lambda i: (i, 0)),
        ],
        out_shape=[
            jax.ShapeDtypeStruct((N_A, 2 * C), f32),
            jax.ShapeDtypeStruct((N_R, 2 * C), f32),
        ],
    )(atom_x, residue_x, W_query, W_key, W_value, W_atom_value,
      row(ln_ain_g), row(ln_ain_b), row(ln_rin_g), row(ln_rin_b))

    mesh = plsc.VectorSubcoreMesh(core_axis_name="c", subcore_axis_name="s")
    gd, gs = pl.kernel(
        _sc_gather,
        out_type=[
            jax.ShapeDtypeStruct((E, 2 * C), f32),
            jax.ShapeDtypeStruct((E, 2 * C), f32),
        ],
        mesh=mesh,
        scratch_types=[
            pltpu.VMEM((2, KG), jnp.int32),
            pltpu.VMEM((2, KG, 2 * C), f32),
            pltpu.SemaphoreType.DMA,
            pltpu.SemaphoreType.DMA,
            pltpu.SemaphoreType.DMA,
            pltpu.SemaphoreType.DMA,
        ],
    )(ta, tr, dst, src)

    e_blocks = E // BLK_E
    score, m = pl.pallas_call(
        _score_body,
        grid=(e_blocks,),
        in_specs=[
            pl.BlockSpec((BLK_E, C), lambda i: (i, 0)),
            pl.BlockSpec((BLK_E, C), lambda i: (i, 0)),
        ],
        out_specs=[
            pl.BlockSpec((BLK_E, H), lambda i: (i, 0)),
            pl.BlockSpec((1, H), lambda i: (0, 0)),
        ],
        out_shape=[
            jax.ShapeDtypeStruct((E, H), f32),
            jax.ShapeDtypeStruct((1, H), f32),
        ],
    )(gd, gs)

    wa, wr, expp = pl.pallas_call(
        _weight_body,
        grid=(e_blocks,),
        in_specs=[
            pl.BlockSpec((BLK_E, H), lambda i: (i, 0)),
            pl.BlockSpec((1, H), lambda i: (0, 0)),
            pl.BlockSpec((BLK_E, C), lambda i: (i, 1)),
            pl.BlockSpec((BLK_E, C), lambda i: (i, 1)),
        ],
        out_specs=[
            pl.BlockSpec((BLK_E, C), lambda i: (i, 0)),
            pl.BlockSpec((BLK_E, C), lambda i: (i, 0)),
            pl.BlockSpec((BLK_E, C // 2), lambda i: (i, 0)),
        ],
        out_shape=[
            jax.ShapeDtypeStruct((E, C), f32),
            jax.ShapeDtypeStruct((E, C), f32),
            jax.ShapeDtypeStruct((E, C // 2), f32),
        ],
    )(score, m, gs, gd)

    z128 = jnp.zeros((N_A, C // 2), f32)
    oa, orr, sex = pl.kernel(
        _sc_scatter,
        out_type=[
            jax.ShapeDtypeStruct((N_A, C), f32),
            jax.ShapeDtypeStruct((N_R, C), f32),
            jax.ShapeDtypeStruct((NC, N_A, C // 2), f32),
        ],
        mesh=mesh,
        scratch_types=[
            pltpu.VMEM((2, KS), jnp.int32),
            pltpu.VMEM((2, KS, C // 2), f32),
            pltpu.VMEM_SHARED((N_A, C // 2), f32),
            pltpu.SemaphoreType.DMA,
            pltpu.SemaphoreType.DMA,
        ],
    )(wa, wr, expp, edge_index.reshape(2 * E), z128)

    return _finish(atom_x, residue_x, oa, orr, sex[0, :, :H],
                   sex[1, :, H:2 * H],
                   ln_aout_g, ln_aout_b, ln_rout_g, ln_rout_b, cm_W1, cm_b1,
                   cm_W2, cm_b2, cm_ln_g, cm_ln_b, rm_W1, rm_b1, rm_W2,
                   rm_b2, rm_ln_g, rm_ln_b)


def _finish(atom_x, residue_x, oa, orr, sab, srb, ln_aout_g,
            ln_aout_b, ln_rout_g, ln_rout_b, cm_W1, cm_b1, cm_W2, cm_b2,
            cm_ln_g, cm_ln_b, rm_W1, rm_b1, rm_W2, rm_b2, rm_ln_g, rm_ln_b):
    f32 = jnp.float32
    n_blocks = N_A // BLK_N
    full = lambda shape: pl.BlockSpec(shape, lambda i: (0, 0))

    def row(v):
        return v.reshape(1, -1)

    def final(x, o, sseg, g_out, b_out, w1, b1, w2, b2, g_m, b_m):
        return pl.pallas_call(
            _final_body,
            grid=(n_blocks,),
            in_specs=[
                pl.BlockSpec((BLK_N, C), lambda i: (i, 0)),
                pl.BlockSpec((BLK_N, C), lambda i: (i, 0)),
                pl.BlockSpec((BLK_N, H), lambda i: (i, 0)),
                full((1, C)), full((1, C)),
                full((2 * C, 2 * C)), full((1, 2 * C)),
                full((2 * C, C)), full((1, C)),
                full((1, C)), full((1, C)),
            ],
            out_specs=pl.BlockSpec((BLK_N, C), lambda i: (i, 0)),
            out_shape=jax.ShapeDtypeStruct((N_A, C), f32),
        )(x, o, sseg, g_out, b_out, w1, b1, w2, b2, g_m, b_m)

    ha = final(atom_x, oa, sab, row(ln_aout_g),
               row(ln_aout_b), cm_W1, row(cm_b1), cm_W2, row(cm_b2),
               row(cm_ln_g), row(cm_ln_b))
    hr = final(residue_x, orr, srb, row(ln_rout_g),
               row(ln_rout_b), rm_W1, row(rm_b1), rm_W2, row(rm_b2),
               row(rm_ln_g), row(rm_ln_b))
    return jnp.concatenate([ha, hr], axis=0)
